# Initial kernel scaffold; baseline (speedup 1.0000x reference)
import jax
import jax.numpy as jnp
from jax.experimental import pallas as pl


def kernel(prev_x, x, labels, times, P):
    raise NotImplementedError("write your pallas kernel here")


# speedup vs baseline: 2.3945x; 2.3945x over previous
import numpy as np
import jax
import jax.numpy as jnp
from jax import lax
from jax.experimental import pallas as pl

_NUM_TIME = 8
_NUM_LABEL = 4
_SPLIT = 6
_D = 32
_NSEG = _NUM_TIME * _NUM_LABEL

_RA = 2000
_RC = 1000


def _seg_body(x_ref, seg_ref, o_ref):
    i = pl.program_id(0)
    xx = x_ref[...]
    sg = seg_ref[...]
    r = xx.shape[0]
    x_aug = jnp.concatenate([xx, jnp.ones((r, 8), jnp.float32)], axis=1)
    ohs = (sg == lax.broadcasted_iota(jnp.int32, (r, _NSEG), 1)).astype(jnp.float32)
    d = lax.dot_general(ohs, x_aug, (((0,), (0,)), ((), ())),
                        preferred_element_type=jnp.float32)

    @pl.when(i == 0)
    def _():
        o_ref[...] = d

    @pl.when(i > 0)
    def _():
        o_ref[...] += d


def _apply_body(x_ref, seg_ref, astack_ref, b_ref, o_ref):
    xx = x_ref[...]
    sg = seg_ref[...]
    r = xx.shape[0]
    c1024 = lax.broadcasted_iota(jnp.int32, (r, _NSEG * _D), 1)
    mask = sg == (c1024 // _D)
    xt = jnp.concatenate([xx] * _NSEG, axis=1)
    zext = jnp.where(mask, xt, 0.0)
    y = lax.dot_general(zext, astack_ref[...], (((1,), (0,)), ((), ())),
                        preferred_element_type=jnp.float32)
    ohs = (sg == lax.broadcasted_iota(jnp.int32, (r, _NSEG), 1)).astype(jnp.float32)
    y = y + lax.dot_general(ohs, b_ref[...], (((1,), (0,)), ((), ())),
                            preferred_element_type=jnp.float32)
    t1 = sg // _NUM_LABEL
    o_ref[...] = jnp.where(t1 < _SPLIT, y, xx)


def _transform_bank(prev_x, labels, P):
    covs = []
    for y in range(_NUM_LABEL):
        mask = (labels == y).astype(prev_x.dtype)
        n = mask.sum()
        mean = (prev_x * mask[:, None]).sum(0) / n
        xc = (prev_x - mean[None, :]) * mask[:, None]
        covs.append(xc.T @ xc / (n - 1.0))
    prev_cov = jnp.stack(covs)

    t = np.arange(_NUM_TIME)
    cond = np.abs(t[None, :] - t[:, None]) > np.minimum(_NUM_TIME - 1 - t, t)[:, None]
    f2 = jnp.asarray(np.where(cond, 2.0, 1.0), dtype=jnp.float32)
    f4 = jnp.asarray(np.where(cond, 4.0, 1.0), dtype=jnp.float32)
    denom = jnp.einsum('atbs,ts->at', P, f2)
    denom2 = denom * denom
    temp = jnp.einsum('atbs,ts->atb', P, f4) / denom2[:, :, None]
    current_cov = jnp.einsum('atb,bij->atij', temp, prev_cov)

    lall, qall = jnp.linalg.eigh(current_cov)
    l_max = lall[:, _NUM_TIME - 1]
    q_max = qall[:, _NUM_TIME - 1]
    left = q_max * jnp.sqrt(l_max)[:, None, :]
    right = qall * (1.0 / jnp.sqrt(lall))[:, :, None, :]
    return jnp.einsum('yik,ytjk->ytij', left, right)


def kernel(prev_x, x, labels, times, P):
    n = x.shape[0]
    seg = (times.astype(jnp.int32) * _NUM_LABEL
           + labels.astype(jnp.int32)).reshape(n, 1)

    a = _transform_bank(prev_x, labels, P)

    o2 = pl.pallas_call(
        _seg_body,
        grid=(n // _RA,),
        in_specs=[
            pl.BlockSpec((_RA, _D), lambda i: (i, 0)),
            pl.BlockSpec((_RA, 1), lambda i: (i, 0)),
        ],
        out_specs=pl.BlockSpec((_NSEG, 40), lambda i: (0, 0)),
        out_shape=jax.ShapeDtypeStruct((_NSEG, 40), jnp.float32),
    )(x, seg)

    mu = o2[:, :_D] / jnp.maximum(o2[:, _D], 1.0)[:, None]
    a_seg = jnp.transpose(a, (1, 0, 2, 3)).reshape(_NSEG, _D, _D)
    b_seg = mu - jnp.einsum('sij,sj->si', a_seg, mu)
    a_stack = jnp.transpose(a_seg, (0, 2, 1)).reshape(_NSEG * _D, _D)

    out = pl.pallas_call(
        _apply_body,
        grid=(n // _RC,),
        in_specs=[
            pl.BlockSpec((_RC, _D), lambda i: (i, 0)),
            pl.BlockSpec((_RC, 1), lambda i: (i, 0)),
            pl.BlockSpec((_NSEG * _D, _D), lambda i: (0, 0)),
            pl.BlockSpec((_NSEG, _D), lambda i: (0, 0)),
        ],
        out_specs=pl.BlockSpec((_RC, _D), lambda i: (i, 0)),
        out_shape=jax.ShapeDtypeStruct((n, _D), jnp.float32),
    )(x, seg, a_stack, b_seg)
    return out


# speedup vs baseline: 2.4580x; 1.0265x over previous
import numpy as np
import jax
import jax.numpy as jnp
from jax import lax
from jax.experimental import pallas as pl

_NUM_TIME = 8
_NUM_LABEL = 4
_SPLIT = 6
_D = 32
_NSEG = _NUM_TIME * _NUM_LABEL

_RA = 5000
_RC = 5000


def _seg_body(x_ref, seg_ref, o_ref):
    i = pl.program_id(0)
    xx = x_ref[...]
    sg = seg_ref[...]
    r = xx.shape[0]
    x_aug = jnp.concatenate([xx, jnp.ones((r, 8), jnp.float32)], axis=1)
    ohs = (sg == lax.broadcasted_iota(jnp.int32, (r, _NSEG), 1)).astype(jnp.float32)
    d = lax.dot_general(ohs, x_aug, (((0,), (0,)), ((), ())),
                        preferred_element_type=jnp.float32)

    @pl.when(i == 0)
    def _():
        o_ref[...] = d

    @pl.when(i > 0)
    def _():
        o_ref[...] += d


def _apply_body(x_ref, seg_ref, astack_ref, b_ref, o_ref):
    xx = x_ref[...]
    sg = seg_ref[...]
    r = xx.shape[0]
    xb = xx.astype(jnp.bfloat16)
    sgb = sg.astype(jnp.bfloat16)
    c1024 = lax.broadcasted_iota(jnp.int32, (r, _NSEG * _D), 1)
    patt = (c1024 // _D).astype(jnp.bfloat16)
    mask = sgb == patt
    xt = jnp.concatenate([xb] * _NSEG, axis=1)
    zext = jnp.where(mask, xt, jnp.bfloat16(0))
    y = lax.dot_general(zext, astack_ref[...], (((1,), (0,)), ((), ())),
                        preferred_element_type=jnp.float32)
    ohs = (sg == lax.broadcasted_iota(jnp.int32, (r, _NSEG), 1)).astype(jnp.float32)
    y = y + lax.dot_general(ohs, b_ref[...], (((1,), (0,)), ((), ())),
                            preferred_element_type=jnp.float32)
    t1 = sg // _NUM_LABEL
    o_ref[...] = jnp.where(t1 < _SPLIT, y, xx)


def _transform_bank(prev_x, labels, P):
    covs = []
    for y in range(_NUM_LABEL):
        mask = (labels == y).astype(prev_x.dtype)
        n = mask.sum()
        mean = (prev_x * mask[:, None]).sum(0) / n
        xc = (prev_x - mean[None, :]) * mask[:, None]
        covs.append(xc.T @ xc / (n - 1.0))
    prev_cov = jnp.stack(covs)

    t = np.arange(_NUM_TIME)
    cond = np.abs(t[None, :] - t[:, None]) > np.minimum(_NUM_TIME - 1 - t, t)[:, None]
    f2 = jnp.asarray(np.where(cond, 2.0, 1.0), dtype=jnp.float32)
    f4 = jnp.asarray(np.where(cond, 4.0, 1.0), dtype=jnp.float32)
    denom = jnp.einsum('atbs,ts->at', P, f2)
    denom2 = denom * denom
    temp = jnp.einsum('atbs,ts->atb', P, f4) / denom2[:, :, None]
    current_cov = jnp.einsum('atb,bij->atij', temp, prev_cov)

    lall, qall = jnp.linalg.eigh(current_cov)
    l_max = lall[:, _NUM_TIME - 1]
    q_max = qall[:, _NUM_TIME - 1]
    left = q_max * jnp.sqrt(l_max)[:, None, :]
    right = qall * (1.0 / jnp.sqrt(lall))[:, :, None, :]
    return jnp.einsum('yik,ytjk->ytij', left, right)


def kernel(prev_x, x, labels, times, P):
    n = x.shape[0]
    seg = (times.astype(jnp.int32) * _NUM_LABEL
           + labels.astype(jnp.int32)).reshape(n, 1)

    a = _transform_bank(prev_x, labels, P)

    o2 = pl.pallas_call(
        _seg_body,
        grid=(n // _RA,),
        in_specs=[
            pl.BlockSpec((_RA, _D), lambda i: (i, 0)),
            pl.BlockSpec((_RA, 1), lambda i: (i, 0)),
        ],
        out_specs=pl.BlockSpec((_NSEG, 40), lambda i: (0, 0)),
        out_shape=jax.ShapeDtypeStruct((_NSEG, 40), jnp.float32),
    )(x, seg)

    mu = o2[:, :_D] / jnp.maximum(o2[:, _D], 1.0)[:, None]
    a_seg = jnp.transpose(a, (1, 0, 2, 3)).reshape(_NSEG, _D, _D)
    b_seg = mu - jnp.einsum('sij,sj->si', a_seg, mu)
    a_stack = jnp.transpose(a_seg, (0, 2, 1)).reshape(
        _NSEG * _D, _D).astype(jnp.bfloat16)

    out = pl.pallas_call(
        _apply_body,
        grid=(n // _RC,),
        in_specs=[
            pl.BlockSpec((_RC, _D), lambda i: (i, 0)),
            pl.BlockSpec((_RC, 1), lambda i: (i, 0)),
            pl.BlockSpec((_NSEG * _D, _D), lambda i: (0, 0)),
            pl.BlockSpec((_NSEG, _D), lambda i: (0, 0)),
        ],
        out_specs=pl.BlockSpec((_RC, _D), lambda i: (i, 0)),
        out_shape=jax.ShapeDtypeStruct((n, _D), jnp.float32),
    )(x, seg, a_stack, b_seg)
    return out
